# baseline (speedup 1.0000x reference)
import jax
import jax.numpy as jnp
from jax.experimental import pallas as pl
from jax.experimental.pallas import tpu as pltpu

_N = 4096
_BR = 512
_W = 1024


def _ia_kernel(ab_ref, a_ref, o_ref):
    g = pl.program_id(0)
    o_ref[...] = jnp.maximum(a_ref[...], 0.0)

    s = jax.nn.sigmoid(ab_ref[0]) + jax.nn.sigmoid(ab_ref[1])
    row0 = g * _BR
    wstart = jnp.clip(row0 - (_W - _BR) // 2, 0, _N - _W)
    wstart = pl.multiple_of(wstart, 256)
    rows = jax.lax.broadcasted_iota(jnp.int32, (_BR, _W), 0) + row0
    cols = jax.lax.broadcasted_iota(jnp.int32, (_BR, _W), 1) + wstart
    d = cols - rows
    corner_s = jnp.where((rows == 0) | (rows == _N - 1), s, 0.0)
    add = (jnp.where(jnp.abs(d) == 1, s, 0.0)
           + jnp.where(d == 0, 1.0 + corner_s, 0.0))
    win = o_ref[:, pl.ds(wstart, _W)]
    o_ref[:, pl.ds(wstart, _W)] = win + add


@jax.jit
def kernel(A, a, b):
    ab = jnp.stack([a, b]).astype(jnp.float32)
    return pl.pallas_call(
        _ia_kernel,
        grid=(_N // _BR,),
        in_specs=[
            pl.BlockSpec(memory_space=pltpu.SMEM),
            pl.BlockSpec((_BR, _N), lambda i: (i, 0)),
        ],
        out_specs=pl.BlockSpec((_BR, _N), lambda i: (i, 0)),
        out_shape=jax.ShapeDtypeStruct((_N, _N), jnp.float32),
    )(ab, A)


# speedup vs baseline: 1.0076x; 1.0076x over previous
import jax
import jax.numpy as jnp
from jax.experimental import pallas as pl
from jax.experimental.pallas import tpu as pltpu

_N = 4096
_BR = 512
_W = 768


def _ia_kernel(ab_ref, a_ref, o_ref):
    g = pl.program_id(0)
    o_ref[...] = jnp.maximum(a_ref[...], 0.0)

    s = jax.nn.sigmoid(ab_ref[0]) + jax.nn.sigmoid(ab_ref[1])
    row0 = g * _BR
    wstart = jnp.clip(row0 - (_W - _BR) // 2, 0, _N - _W)
    wstart = pl.multiple_of(wstart, 128)
    rows = jax.lax.broadcasted_iota(jnp.int32, (_BR, _W), 0) + row0
    cols = jax.lax.broadcasted_iota(jnp.int32, (_BR, _W), 1) + wstart
    d = cols - rows
    corner_s = jnp.where((rows == 0) | (rows == _N - 1), s, 0.0)
    add = (jnp.where(jnp.abs(d) == 1, s, 0.0)
           + jnp.where(d == 0, 1.0 + corner_s, 0.0))
    win = o_ref[:, pl.ds(wstart, _W)]
    o_ref[:, pl.ds(wstart, _W)] = win + add


@jax.jit
def kernel(A, a, b):
    ab = jnp.stack([a, b]).astype(jnp.float32)
    return pl.pallas_call(
        _ia_kernel,
        grid=(_N // _BR,),
        in_specs=[
            pl.BlockSpec(memory_space=pltpu.SMEM),
            pl.BlockSpec((_BR, _N), lambda i: (i, 0)),
        ],
        out_specs=pl.BlockSpec((_BR, _N), lambda i: (i, 0)),
        out_shape=jax.ShapeDtypeStruct((_N, _N), jnp.float32),
    )(ab, A)


# speedup vs baseline: 1.0380x; 1.0302x over previous
import jax
import jax.numpy as jnp
from jax.experimental import pallas as pl
from jax.experimental.pallas import tpu as pltpu

_N = 4096
_BR = 512


def _ia_kernel(ab_ref, a_ref, o_ref):
    g = pl.program_id(0)
    o_ref[...] = jnp.maximum(a_ref[...], 0.0)

    s = jax.nn.sigmoid(ab_ref[0]) + jax.nn.sigmoid(ab_ref[1])
    wstart = pl.multiple_of(g * _BR, _BR)
    ci = jax.lax.broadcasted_iota(jnp.int32, (_BR, _BR), 0)
    cj = jax.lax.broadcasted_iota(jnp.int32, (_BR, _BR), 1)
    d = cj - ci
    add = jnp.where(jnp.abs(d) == 1, s, 0.0) + jnp.where(d == 0, 1.0, 0.0)
    o_ref[:, pl.ds(wstart, _BR)] = o_ref[:, pl.ds(wstart, _BR)] + add

    last = _N // _BR - 1
    lanes = jax.lax.broadcasted_iota(jnp.int32, (1, 128), 1)

    def _patch(row, base, lane):
        base = pl.multiple_of(base, 128)
        cur = o_ref[row:row + 1, pl.ds(base, 128)]
        o_ref[row:row + 1, pl.ds(base, 128)] = (
            cur + jnp.where(lanes == lane, s, 0.0))

    @pl.when(g > 0)
    def _():
        _patch(0, wstart - 128, 127)

    @pl.when(g < last)
    def _():
        _patch(_BR - 1, wstart + _BR, 0)

    @pl.when(g == 0)
    def _():
        _patch(0, 0, 0)

    @pl.when(g == last)
    def _():
        _patch(_BR - 1, _N - 128, 127)


@jax.jit
def kernel(A, a, b):
    ab = jnp.stack([a, b]).astype(jnp.float32)
    return pl.pallas_call(
        _ia_kernel,
        grid=(_N // _BR,),
        in_specs=[
            pl.BlockSpec(memory_space=pltpu.SMEM),
            pl.BlockSpec((_BR, _N), lambda i: (i, 0)),
        ],
        out_specs=pl.BlockSpec((_BR, _N), lambda i: (i, 0)),
        out_shape=jax.ShapeDtypeStruct((_N, _N), jnp.float32),
    )(ab, A)
